# baseline (speedup 1.0000x reference)
import functools

import jax
import jax.numpy as jnp
from jax import lax
from jax.experimental import pallas as pl
from jax.experimental.pallas import tpu as pltpu
from jax.experimental.pallas import tpu_sc as plsc

_POOLED = 7
_SCALE = 0.0625
_GRID = 2
_K = 512
_C = 256
_B = 2
_H = 50
_W = 50
_T = _POOLED * _POOLED * _GRID * _GRID
_NW = 32
_CHUNKS = _K * _POOLED
_CPW = _CHUNKS // _NW
_CW = _POOLED * _GRID * _GRID * 4


def _prep_body(rois_ref, idx_r, w_r, oidx_r):
    r = rois_ref[:, :]
    b = r[:, 0:1].astype(jnp.int32)
    sw = r[:, 1:2] * _SCALE - 0.5
    sh = r[:, 2:3] * _SCALE - 0.5
    ew = r[:, 3:4] * _SCALE - 0.5
    eh = r[:, 4:5] * _SCALE - 0.5
    bin_w = (ew - sw) / _POOLED
    bin_h = (eh - sh) / _POOLED

    u = lax.broadcasted_iota(jnp.int32, (1, _T * 4), 1)
    corner = u % 4
    t = u // 4
    ph = (t // 28).astype(jnp.float32)
    pw = ((t // 4) % 7).astype(jnp.float32)
    iy = ((t % 4) // 2).astype(jnp.float32)
    ix = (t % 2).astype(jnp.float32)

    y = sh + ph * bin_h + (iy + 0.5) * bin_h / _GRID
    x = sw + pw * bin_w + (ix + 0.5) * bin_w / _GRID
    valid = ((y >= -1.0) & (y <= float(_H)) &
             (x >= -1.0) & (x <= float(_W)))
    y = jnp.maximum(y, 0.0)
    x = jnp.maximum(x, 0.0)
    y_low0 = jnp.floor(y).astype(jnp.int32)
    x_low0 = jnp.floor(x).astype(jnp.int32)
    hi_y = y_low0 >= _H - 1
    hi_x = x_low0 >= _W - 1
    y_low = jnp.where(hi_y, _H - 1, y_low0)
    x_low = jnp.where(hi_x, _W - 1, x_low0)
    y_high = jnp.where(hi_y, _H - 1, y_low0 + 1)
    x_high = jnp.where(hi_x, _W - 1, x_low0 + 1)
    ylf = y_low.astype(jnp.float32)
    xlf = x_low.astype(jnp.float32)
    ly = jnp.where(hi_y, 0.0, y - ylf)
    lx = jnp.where(hi_x, 0.0, x - xlf)
    hy = 1.0 - ly
    hx = 1.0 - lx
    vm = jnp.where(valid, 0.25, 0.0)
    cy = jnp.where(corner < 2, hy, ly)
    cx = jnp.where(corner % 2 == 0, hx, lx)
    w_r[:, :] = cy * cx * vm

    u2 = lax.broadcasted_iota(jnp.int32, (1, _T * 2), 1)
    lh = u2 % 2
    t2 = u2 // 2
    ph2 = (t2 // 28).astype(jnp.float32)
    pw2 = ((t2 // 4) % 7).astype(jnp.float32)
    iy2 = ((t2 % 4) // 2).astype(jnp.float32)
    ix2 = (t2 % 2).astype(jnp.float32)
    y2 = sh + ph2 * bin_h + (iy2 + 0.5) * bin_h / _GRID
    x2 = sw + pw2 * bin_w + (ix2 + 0.5) * bin_w / _GRID
    y2 = jnp.maximum(y2, 0.0)
    x2 = jnp.maximum(x2, 0.0)
    yl0 = jnp.floor(y2).astype(jnp.int32)
    xl0 = jnp.floor(x2).astype(jnp.int32)
    yl = jnp.minimum(yl0, _H - 1)
    xl = jnp.minimum(xl0, _W - 1)
    yh = jnp.minimum(yl0 + 1, _H - 1)
    gy2 = jnp.where(lh == 0, yl, yh)
    f = b * (_H * _W) + gy2 * _W + xl
    idx_r[:, :] = jnp.where(f % 2 == 0, f // 2, (_B * _H * _W) // 2 + f // 2)

    k = lax.broadcasted_iota(jnp.int32, (_K, 1), 0)
    o = lax.broadcasted_iota(jnp.int32, (1, 2 * _POOLED * _POOLED), 1)
    oidx_r[:, :] = (o // 2) * 1024 + (k // 8) * 16 + (o % 2) * 8 + (k % 8)


_prep = pl.pallas_call(
    _prep_body,
    out_shape=(
        jax.ShapeDtypeStruct((_K, _T * 2), jnp.int32),
        jax.ShapeDtypeStruct((_K, _T * 4), jnp.float32),
        jax.ShapeDtypeStruct((_K, 2 * _POOLED * _POOLED), jnp.int32),
    ),
)


def _transpose_body(x_ref, t_ref):
    t_ref[0] = jnp.transpose(x_ref[0], (1, 0)).astype(jnp.bfloat16)


_transpose = pl.pallas_call(
    _transpose_body,
    grid=(_B,),
    in_specs=[pl.BlockSpec((1, _C, _H * _W), lambda i: (i, 0, 0))],
    out_specs=pl.BlockSpec((1, _H * _W, _C), lambda i: (i, 0, 0)),
    out_shape=jax.ShapeDtypeStruct((_B, _H * _W, _C), jnp.bfloat16),
)


_RPW = _K // _NW
_OROI = _C * _POOLED * _POOLED


_ORR = 2 * _POOLED * _POOLED


def _sc_body(table_h, idx_h, w_h, oidx_h, out_h,
             idx_all, w_all, oidx_all, rows0, rows1, out_v,
             sem0, sem1, osem0, osem1):
    cid = lax.axis_index("c")
    sid = lax.axis_index("s")
    wid = sid * 2 + cid
    base_ch = wid * _CPW
    base_k = wid * _RPW

    pltpu.sync_copy(idx_h.at[pl.ds(base_ch, _CPW)], idx_all)
    pltpu.sync_copy(w_h.at[pl.ds(base_ch, _CPW)], w_all)
    pltpu.sync_copy(oidx_h.at[pl.ds(base_k, _RPW)], oidx_all)

    def issue(c, rows_b, sem_b):
        pltpu.async_copy(table_h.at[idx_all.at[c]], rows_b, sem_b)

    def wait(c, rows_b, sem_b):
        pltpu.make_async_copy(table_h.at[idx_all.at[c]], rows_b, sem_b).wait()

    def out_issue(r, par, osem):
        pltpu.async_copy(out_v.at[pl.ds(par * _ORR, _ORR)],
                         out_h.at[oidx_all.at[r]], osem)

    def out_wait(r, par, osem):
        pltpu.make_async_copy(out_v.at[pl.ds(par * _ORR, _ORR)],
                              out_h.at[oidx_all.at[r]], osem).wait()

    def compute(c, rows_b):
        r = c // 7
        ph = c % 7
        par = r % 2

        @pl.when((ph == 0) & (r >= 2))
        def _():
            @pl.when(par == 0)
            def _():
                out_wait(r - 2, 0, osem0)

            @pl.when(par == 1)
            def _():
                out_wait(r - 2, 1, osem1)

        def cell_body(cell, carry2):
            cbase = cell * 16
            wvec = w_all[c, pl.ds(cbase, 16)]
            acc_e = [jnp.zeros((16,), jnp.float32) for _ in range(8)]
            acc_o = [jnp.zeros((16,), jnp.float32) for _ in range(8)]
            cbase2 = cell * 8
            for s in range(4):
                wbs = []
                for ci in range(4):
                    wv16 = jnp.broadcast_to(wvec[s * 4 + ci], (16,))
                    wbs.append(plsc.pack(wv16, wv16,
                                         format=plsc.PackFormat.INTERLEAVED))
                b0 = cbase2 + s * 2
                for e in range(8):
                    pacc = (rows_b[b0, pl.ds(e * 32, 32)] * wbs[0]
                            + rows_b[b0, pl.ds(_C + e * 32, 32)] * wbs[1]
                            + rows_b[b0 + 1, pl.ds(e * 32, 32)] * wbs[2]
                            + rows_b[b0 + 1, pl.ds(_C + e * 32, 32)] * wbs[3])
                    pe, po = plsc.unpack(pacc,
                                         format=plsc.PackFormat.INTERLEAVED,
                                         preferred_element_type=jnp.float32)
                    acc_e[e] = acc_e[e] + pe
                    acc_o[e] = acc_o[e] + po
            rbase = par * _ORR + (ph * 7 + cell) * 2
            for e in range(8):
                out_v[rbase + e // 4, pl.ds((e % 4) * 32, 16)] = acc_e[e]
                out_v[rbase + e // 4, pl.ds((e % 4) * 32 + 16, 16)] = acc_o[e]
            return carry2

        lax.fori_loop(0, _POOLED, cell_body, 0, unroll=True)

        @pl.when(ph == 6)
        def _():
            @pl.when(par == 0)
            def _():
                out_issue(r, 0, osem0)

            @pl.when(par == 1)
            def _():
                out_issue(r, 1, osem1)

    issue(0, rows0, sem0)

    def pair_body(p, carry):
        c0 = 2 * p
        issue(c0 + 1, rows1, sem1)
        wait(c0, rows0, sem0)
        compute(c0, rows0)

        @pl.when(c0 + 2 < _CPW)
        def _():
            issue(c0 + 2, rows0, sem0)

        wait(c0 + 1, rows1, sem1)
        compute(c0 + 1, rows1)
        return carry

    lax.fori_loop(0, _CPW // 2, pair_body, 0, unroll=False)

    out_wait(_RPW - 2, 0, osem0)
    out_wait(_RPW - 1, 1, osem1)


@functools.lru_cache(maxsize=None)
def _get_sc_gather():
    return functools.partial(
        pl.kernel,
        out_type=jax.ShapeDtypeStruct((_K * _ORR, 128), jnp.float32),
        mesh=plsc.VectorSubcoreMesh(core_axis_name="c", subcore_axis_name="s"),
        scratch_types=[
            pltpu.VMEM((_CPW, _CW // 2), jnp.int32),
            pltpu.VMEM((_CPW, _CW), jnp.float32),
            pltpu.VMEM((_RPW, _ORR), jnp.int32),
            pltpu.VMEM((_CW // 2, 2 * _C), jnp.bfloat16),
            pltpu.VMEM((_CW // 2, 2 * _C), jnp.bfloat16),
            pltpu.VMEM((2 * _ORR, 128), jnp.float32),
            pltpu.SemaphoreType.DMA,
            pltpu.SemaphoreType.DMA,
            pltpu.SemaphoreType.DMA,
            pltpu.SemaphoreType.DMA,
        ],
        compiler_params=pltpu.CompilerParams(use_tc_tiling_on_sc=False,
                                             needs_layout_passes=False),
    )(_sc_body)


def kernel(input, rois):
    perm = jnp.arange(_C).reshape(_C // 32, 2, 16).transpose(0, 2, 1).reshape(
        _C)
    xp = jnp.take(input, perm, axis=1)
    table = _transpose(xp.reshape(_B, _C, _H * _W)).reshape(_B * _H * _W, _C)
    pad = jnp.zeros((1, _C), jnp.bfloat16)
    tab2 = jnp.concatenate([
        table.reshape(_B * _H * _W // 2, 2 * _C),
        jnp.concatenate([table[1:], pad], axis=0).reshape(
            _B * _H * _W // 2, 2 * _C),
    ], axis=0)
    idx, w, oidx = _prep(rois)
    idx = idx.reshape(_CHUNKS, _CW // 2)
    w = w.reshape(_CHUNKS, _CW)
    buf = _get_sc_gather()(tab2, idx, w, oidx)
    b = buf.reshape(_POOLED * _POOLED, _K // 8, _C // 128, 8, 128)
    out = b.transpose(1, 3, 2, 4, 0).reshape(_K, _C, _POOLED * _POOLED)
    return out.reshape(_K, _C, _POOLED, _POOLED)


# speedup vs baseline: 1.2471x; 1.2471x over previous
import functools

import jax
import jax.numpy as jnp
from jax import lax
from jax.experimental import pallas as pl
from jax.experimental.pallas import tpu as pltpu
from jax.experimental.pallas import tpu_sc as plsc

_POOLED = 7
_SCALE = 0.0625
_GRID = 2
_K = 512
_C = 256
_B = 2
_H = 50
_W = 50
_T = _POOLED * _POOLED * _GRID * _GRID
_NW = 32
_CHUNKS = _K * _POOLED
_CPW = _CHUNKS // _NW
_CW = _POOLED * _GRID * _GRID * 4


def _prep_body(rois_ref, idx_r, w_r, oidx_r):
    r = rois_ref[:, :]
    b = r[:, 0:1].astype(jnp.int32)
    sw = r[:, 1:2] * _SCALE - 0.5
    sh = r[:, 2:3] * _SCALE - 0.5
    ew = r[:, 3:4] * _SCALE - 0.5
    eh = r[:, 4:5] * _SCALE - 0.5
    bin_w = (ew - sw) / _POOLED
    bin_h = (eh - sh) / _POOLED

    u = lax.broadcasted_iota(jnp.int32, (1, _T * 4), 1)
    corner = u % 4
    t = u // 4
    ph = (t // 28).astype(jnp.float32)
    pw = ((t // 4) % 7).astype(jnp.float32)
    iy = ((t % 4) // 2).astype(jnp.float32)
    ix = (t % 2).astype(jnp.float32)

    y = sh + ph * bin_h + (iy + 0.5) * bin_h / _GRID
    x = sw + pw * bin_w + (ix + 0.5) * bin_w / _GRID
    valid = ((y >= -1.0) & (y <= float(_H)) &
             (x >= -1.0) & (x <= float(_W)))
    y = jnp.maximum(y, 0.0)
    x = jnp.maximum(x, 0.0)
    y_low0 = jnp.floor(y).astype(jnp.int32)
    x_low0 = jnp.floor(x).astype(jnp.int32)
    hi_y = y_low0 >= _H - 1
    hi_x = x_low0 >= _W - 1
    y_low = jnp.where(hi_y, _H - 1, y_low0)
    x_low = jnp.where(hi_x, _W - 1, x_low0)
    y_high = jnp.where(hi_y, _H - 1, y_low0 + 1)
    x_high = jnp.where(hi_x, _W - 1, x_low0 + 1)
    ylf = y_low.astype(jnp.float32)
    xlf = x_low.astype(jnp.float32)
    ly = jnp.where(hi_y, 0.0, y - ylf)
    lx = jnp.where(hi_x, 0.0, x - xlf)
    hy = 1.0 - ly
    hx = 1.0 - lx
    vm = jnp.where(valid, 0.25, 0.0)
    cy = jnp.where(corner < 2, hy, ly)
    cx = jnp.where(corner % 2 == 0, hx, lx)
    w_r[:, :] = cy * cx * vm
    gy = jnp.where(corner < 2, y_low, y_high)
    gx = jnp.where(corner % 2 == 0, x_low, x_high)
    idx_r[:, :] = b * (_H * _W) + gy * _W + gx

    k = lax.broadcasted_iota(jnp.int32, (_K, 1), 0)
    o = lax.broadcasted_iota(jnp.int32, (1, 2 * _POOLED * _POOLED), 1)
    oidx_r[:, :] = (o // 2) * 1024 + (k // 8) * 16 + (o % 2) * 8 + (k % 8)


_prep = pl.pallas_call(
    _prep_body,
    out_shape=(
        jax.ShapeDtypeStruct((_K, _T * 4), jnp.int32),
        jax.ShapeDtypeStruct((_K, _T * 4), jnp.float32),
        jax.ShapeDtypeStruct((_K, 2 * _POOLED * _POOLED), jnp.int32),
    ),
)


def _transpose_body(x_ref, t_ref):
    t_ref[0] = jnp.transpose(x_ref[0], (1, 0)).astype(jnp.bfloat16)


_transpose = pl.pallas_call(
    _transpose_body,
    grid=(_B,),
    in_specs=[pl.BlockSpec((1, _C, _H * _W), lambda i: (i, 0, 0))],
    out_specs=pl.BlockSpec((1, _H * _W, _C), lambda i: (i, 0, 0)),
    out_shape=jax.ShapeDtypeStruct((_B, _H * _W, _C), jnp.bfloat16),
)


_RPW = _K // _NW
_OROI = _C * _POOLED * _POOLED


_ORR = 2 * _POOLED * _POOLED


def _sc_body(table_h, idx_h, w_h, oidx_h, out_h,
             idx_all, w_all, oidx_all, rows0, rows1, out_v,
             sem0, sem1, osem0, osem1):
    cid = lax.axis_index("c")
    sid = lax.axis_index("s")
    wid = sid * 2 + cid
    base_ch = wid * _CPW
    base_k = wid * _RPW

    pltpu.sync_copy(idx_h.at[pl.ds(base_ch, _CPW)], idx_all)
    pltpu.sync_copy(w_h.at[pl.ds(base_ch, _CPW)], w_all)
    pltpu.sync_copy(oidx_h.at[pl.ds(base_k, _RPW)], oidx_all)

    def issue(c, rows_b, sem_b):
        pltpu.async_copy(table_h.at[idx_all.at[c]], rows_b, sem_b)

    def wait(c, rows_b, sem_b):
        pltpu.make_async_copy(table_h.at[idx_all.at[c]], rows_b, sem_b).wait()

    def out_issue(r, par, osem):
        pltpu.async_copy(out_v.at[pl.ds(par * _ORR, _ORR)],
                         out_h.at[oidx_all.at[r]], osem)

    def out_wait(r, par, osem):
        pltpu.make_async_copy(out_v.at[pl.ds(par * _ORR, _ORR)],
                              out_h.at[oidx_all.at[r]], osem).wait()

    def compute(c, rows_b):
        r = c // 7
        ph = c % 7
        par = r % 2

        @pl.when((ph == 0) & (r >= 2))
        def _():
            @pl.when(par == 0)
            def _():
                out_wait(r - 2, 0, osem0)

            @pl.when(par == 1)
            def _():
                out_wait(r - 2, 1, osem1)

        def cell_body(cell, carry2):
            cbase = cell * 16
            wvec = w_all[c, pl.ds(cbase, 16)]
            acc_e = [jnp.zeros((16,), jnp.float32) for _ in range(8)]
            acc_o = [jnp.zeros((16,), jnp.float32) for _ in range(8)]
            for s in range(4):
                wbs = []
                for ci in range(4):
                    wv16 = jnp.broadcast_to(wvec[s * 4 + ci], (16,))
                    wbs.append(plsc.pack(wv16, wv16,
                                         format=plsc.PackFormat.INTERLEAVED))
                for e in range(8):
                    pacc = None
                    for ci in range(4):
                        rv = rows_b[cbase + s * 4 + ci, pl.ds(e * 32, 32)]
                        prod = rv * wbs[ci]
                        pacc = prod if pacc is None else pacc + prod
                    pe, po = plsc.unpack(pacc,
                                         format=plsc.PackFormat.INTERLEAVED,
                                         preferred_element_type=jnp.float32)
                    acc_e[e] = acc_e[e] + pe
                    acc_o[e] = acc_o[e] + po
            rbase = par * _ORR + (ph * 7 + cell) * 2
            for e in range(8):
                out_v[rbase + e // 4, pl.ds((e % 4) * 32, 16)] = acc_e[e]
                out_v[rbase + e // 4, pl.ds((e % 4) * 32 + 16, 16)] = acc_o[e]
            return carry2

        lax.fori_loop(0, _POOLED, cell_body, 0, unroll=2)

        @pl.when(ph == 6)
        def _():
            @pl.when(par == 0)
            def _():
                out_issue(r, 0, osem0)

            @pl.when(par == 1)
            def _():
                out_issue(r, 1, osem1)

    issue(0, rows0, sem0)

    def pair_body(p, carry):
        c0 = 2 * p
        issue(c0 + 1, rows1, sem1)
        wait(c0, rows0, sem0)
        compute(c0, rows0)

        @pl.when(c0 + 2 < _CPW)
        def _():
            issue(c0 + 2, rows0, sem0)

        wait(c0 + 1, rows1, sem1)
        compute(c0 + 1, rows1)
        return carry

    lax.fori_loop(0, _CPW // 2, pair_body, 0, unroll=False)

    out_wait(_RPW - 2, 0, osem0)
    out_wait(_RPW - 1, 1, osem1)


@functools.lru_cache(maxsize=None)
def _get_sc_gather():
    return functools.partial(
        pl.kernel,
        out_type=jax.ShapeDtypeStruct((_K * _ORR, 128), jnp.float32),
        mesh=plsc.VectorSubcoreMesh(core_axis_name="c", subcore_axis_name="s"),
        scratch_types=[
            pltpu.VMEM((_CPW, _CW), jnp.int32),
            pltpu.VMEM((_CPW, _CW), jnp.float32),
            pltpu.VMEM((_RPW, _ORR), jnp.int32),
            pltpu.VMEM((_CW, _C), jnp.bfloat16),
            pltpu.VMEM((_CW, _C), jnp.bfloat16),
            pltpu.VMEM((2 * _ORR, 128), jnp.float32),
            pltpu.SemaphoreType.DMA,
            pltpu.SemaphoreType.DMA,
            pltpu.SemaphoreType.DMA,
            pltpu.SemaphoreType.DMA,
        ],
        compiler_params=pltpu.CompilerParams(use_tc_tiling_on_sc=False,
                                             needs_layout_passes=False),
    )(_sc_body)


def kernel(input, rois):
    perm = jnp.arange(_C).reshape(_C // 32, 2, 16).transpose(0, 2, 1).reshape(
        _C)
    xp = jnp.take(input, perm, axis=1)
    table = _transpose(xp.reshape(_B, _C, _H * _W)).reshape(_B * _H * _W, _C)
    idx, w, oidx = _prep(rois)
    idx = idx.reshape(_CHUNKS, _CW)
    w = w.reshape(_CHUNKS, _CW)
    buf = _get_sc_gather()(table, idx, w, oidx)
    b = buf.reshape(_POOLED * _POOLED, _K // 8, _C // 128, 8, 128)
    out = b.transpose(1, 3, 2, 4, 0).reshape(_K, _C, _POOLED * _POOLED)
    return out.reshape(_K, _C, _POOLED, _POOLED)


# speedup vs baseline: 1.2510x; 1.0031x over previous
import functools

import jax
import jax.numpy as jnp
from jax import lax
from jax.experimental import pallas as pl
from jax.experimental.pallas import tpu as pltpu
from jax.experimental.pallas import tpu_sc as plsc

_POOLED = 7
_SCALE = 0.0625
_GRID = 2
_K = 512
_C = 256
_B = 2
_H = 50
_W = 50
_T = _POOLED * _POOLED * _GRID * _GRID
_NW = 32
_CHUNKS = _K * _POOLED
_CPW = _CHUNKS // _NW
_CW = _POOLED * _GRID * _GRID * 4


def _prep_body(rois7_ref, idx_r, w_r, oidx_r):
    r = rois7_ref[:, :]
    b = r[:, 0:1].astype(jnp.int32)
    sw = r[:, 1:2] * _SCALE - 0.5
    sh = r[:, 2:3] * _SCALE - 0.5
    ew = r[:, 3:4] * _SCALE - 0.5
    eh = r[:, 4:5] * _SCALE - 0.5
    bin_w = (ew - sw) / _POOLED
    bin_h = (eh - sh) / _POOLED

    row = lax.broadcasted_iota(jnp.int32, (_CHUNKS, 1), 0)
    col = lax.broadcasted_iota(jnp.int32, (1, 128), 1)
    colc = jnp.minimum(col, _CW - 1)

    corner = colc % 4
    ph = (row % _POOLED).astype(jnp.float32)
    pw = (colc // 16).astype(jnp.float32)
    iy = ((colc % 16) // 8).astype(jnp.float32)
    ix = ((colc % 8) // 4).astype(jnp.float32)

    y = sh + ph * bin_h + (iy + 0.5) * bin_h / _GRID
    x = sw + pw * bin_w + (ix + 0.5) * bin_w / _GRID
    valid = ((y >= -1.0) & (y <= float(_H)) &
             (x >= -1.0) & (x <= float(_W)))
    y = jnp.maximum(y, 0.0)
    x = jnp.maximum(x, 0.0)
    y_low0 = jnp.floor(y).astype(jnp.int32)
    x_low0 = jnp.floor(x).astype(jnp.int32)
    hi_y = y_low0 >= _H - 1
    hi_x = x_low0 >= _W - 1
    y_low = jnp.where(hi_y, _H - 1, y_low0)
    x_low = jnp.where(hi_x, _W - 1, x_low0)
    y_high = jnp.where(hi_y, _H - 1, y_low0 + 1)
    x_high = jnp.where(hi_x, _W - 1, x_low0 + 1)
    ylf = y_low.astype(jnp.float32)
    xlf = x_low.astype(jnp.float32)
    ly = jnp.where(hi_y, 0.0, y - ylf)
    lx = jnp.where(hi_x, 0.0, x - xlf)
    hy = 1.0 - ly
    hx = 1.0 - lx
    vm = jnp.where(valid, 0.25, 0.0)
    cy = jnp.where(corner < 2, hy, ly)
    cx = jnp.where(corner % 2 == 0, hx, lx)
    w_r[:, :] = cy * cx * vm
    gy = jnp.where(corner < 2, y_low, y_high)
    gx = jnp.where(corner % 2 == 0, x_low, x_high)
    idx_r[:, :] = b * (_H * _W) + gy * _W + gx

    k = lax.broadcasted_iota(jnp.int32, (_K, 1), 0)
    o = lax.broadcasted_iota(jnp.int32, (1, 2 * _POOLED * _POOLED), 1)
    oidx_r[:, :] = (o // 2) * 1024 + (k // 8) * 16 + (o % 2) * 8 + (k % 8)


_prep = pl.pallas_call(
    _prep_body,
    out_shape=(
        jax.ShapeDtypeStruct((_CHUNKS, 128), jnp.int32),
        jax.ShapeDtypeStruct((_CHUNKS, 128), jnp.float32),
        jax.ShapeDtypeStruct((_K, 2 * _POOLED * _POOLED), jnp.int32),
    ),
)


def _transpose_body(x_ref, t_ref):
    t_ref[0] = jnp.transpose(x_ref[0], (1, 0)).astype(jnp.bfloat16)


_transpose = pl.pallas_call(
    _transpose_body,
    grid=(_B,),
    in_specs=[pl.BlockSpec((1, _C, _H * _W), lambda i: (i, 0, 0))],
    out_specs=pl.BlockSpec((1, _H * _W, _C), lambda i: (i, 0, 0)),
    out_shape=jax.ShapeDtypeStruct((_B, _H * _W, _C), jnp.bfloat16),
)


_RPW = _K // _NW
_OROI = _C * _POOLED * _POOLED


_ORR = 2 * _POOLED * _POOLED


def _sc_body(table_h, idx_h, w_h, oidx_h, out_h,
             idx_all, w_all, oidx_all, rows0, rows1, out_v,
             sem0, sem1, osem0, osem1):
    cid = lax.axis_index("c")
    sid = lax.axis_index("s")
    wid = sid * 2 + cid
    base_ch = wid * _CPW
    base_k = wid * _RPW

    pltpu.sync_copy(idx_h.at[pl.ds(base_ch, _CPW)], idx_all)
    pltpu.sync_copy(w_h.at[pl.ds(base_ch, _CPW)], w_all)
    pltpu.sync_copy(oidx_h.at[pl.ds(base_k, _RPW)], oidx_all)

    def issue(c, rows_b, sem_b):
        pltpu.async_copy(table_h.at[idx_all.at[c, pl.ds(0, _CW)]],
                         rows_b, sem_b)

    def wait(c, rows_b, sem_b):
        pltpu.make_async_copy(table_h.at[idx_all.at[c, pl.ds(0, _CW)]],
                              rows_b, sem_b).wait()

    def out_issue(r, par, osem):
        pltpu.async_copy(out_v.at[pl.ds(par * _ORR, _ORR)],
                         out_h.at[oidx_all.at[r]], osem)

    def out_wait(r, par, osem):
        pltpu.make_async_copy(out_v.at[pl.ds(par * _ORR, _ORR)],
                              out_h.at[oidx_all.at[r]], osem).wait()

    def compute(c, rows_b):
        r = c // 7
        ph = c % 7
        par = r % 2

        @pl.when((ph == 0) & (r >= 2))
        def _():
            @pl.when(par == 0)
            def _():
                out_wait(r - 2, 0, osem0)

            @pl.when(par == 1)
            def _():
                out_wait(r - 2, 1, osem1)

        def cell_body(cell, carry2):
            cbase = cell * 16
            wvec = w_all[c, pl.ds(cbase, 16)]
            acc_e = [jnp.zeros((16,), jnp.float32) for _ in range(8)]
            acc_o = [jnp.zeros((16,), jnp.float32) for _ in range(8)]
            for s in range(4):
                wbs = []
                for ci in range(4):
                    wv16 = jnp.broadcast_to(wvec[s * 4 + ci], (16,))
                    wbs.append(plsc.pack(wv16, wv16,
                                         format=plsc.PackFormat.INTERLEAVED))
                for e in range(8):
                    pacc = None
                    for ci in range(4):
                        rv = rows_b[cbase + s * 4 + ci, pl.ds(e * 32, 32)]
                        prod = rv * wbs[ci]
                        pacc = prod if pacc is None else pacc + prod
                    pe, po = plsc.unpack(pacc,
                                         format=plsc.PackFormat.INTERLEAVED,
                                         preferred_element_type=jnp.float32)
                    acc_e[e] = acc_e[e] + pe
                    acc_o[e] = acc_o[e] + po
            rbase = par * _ORR + (ph * 7 + cell) * 2
            for e in range(8):
                out_v[rbase + e // 4, pl.ds((e % 4) * 32, 16)] = acc_e[e]
                out_v[rbase + e // 4, pl.ds((e % 4) * 32 + 16, 16)] = acc_o[e]
            return carry2

        lax.fori_loop(0, _POOLED, cell_body, 0, unroll=2)

        @pl.when(ph == 6)
        def _():
            @pl.when(par == 0)
            def _():
                out_issue(r, 0, osem0)

            @pl.when(par == 1)
            def _():
                out_issue(r, 1, osem1)

    issue(0, rows0, sem0)

    def pair_body(p, carry):
        c0 = 2 * p
        issue(c0 + 1, rows1, sem1)
        wait(c0, rows0, sem0)
        compute(c0, rows0)

        @pl.when(c0 + 2 < _CPW)
        def _():
            issue(c0 + 2, rows0, sem0)

        wait(c0 + 1, rows1, sem1)
        compute(c0 + 1, rows1)
        return carry

    lax.fori_loop(0, _CPW // 2, pair_body, 0, unroll=False)

    out_wait(_RPW - 2, 0, osem0)
    out_wait(_RPW - 1, 1, osem1)


@functools.lru_cache(maxsize=None)
def _get_sc_gather():
    return functools.partial(
        pl.kernel,
        out_type=jax.ShapeDtypeStruct((_K * _ORR, 128), jnp.float32),
        mesh=plsc.VectorSubcoreMesh(core_axis_name="c", subcore_axis_name="s"),
        scratch_types=[
            pltpu.VMEM((_CPW, 128), jnp.int32),
            pltpu.VMEM((_CPW, 128), jnp.float32),
            pltpu.VMEM((_RPW, _ORR), jnp.int32),
            pltpu.VMEM((_CW, _C), jnp.bfloat16),
            pltpu.VMEM((_CW, _C), jnp.bfloat16),
            pltpu.VMEM((2 * _ORR, 128), jnp.float32),
            pltpu.SemaphoreType.DMA,
            pltpu.SemaphoreType.DMA,
            pltpu.SemaphoreType.DMA,
            pltpu.SemaphoreType.DMA,
        ],
        compiler_params=pltpu.CompilerParams(use_tc_tiling_on_sc=False,
                                             needs_layout_passes=False),
    )(_sc_body)


def kernel(input, rois):
    rois7 = jnp.repeat(rois, _POOLED, axis=0)
    perm = jnp.arange(_C).reshape(_C // 32, 2, 16).transpose(0, 2, 1).reshape(
        _C)
    xp = jnp.take(input, perm, axis=1)
    table = _transpose(xp.reshape(_B, _C, _H * _W)).reshape(_B * _H * _W, _C)
    idx, w, oidx = _prep(rois7)
    buf = _get_sc_gather()(table, idx, w, oidx)
    b = buf.reshape(_POOLED * _POOLED, _K // 8, _C // 128, 8, 128)
    out = b.transpose(1, 3, 2, 4, 0).reshape(_K, _C, _POOLED * _POOLED)
    return out.reshape(_K, _C, _POOLED, _POOLED)
